# baseline (speedup 1.0000x reference)
import jax
import jax.numpy as jnp
from jax import lax
from jax.experimental import pallas as pl
from jax.experimental.pallas import tpu as pltpu

N, D, E, OUT = 2048, 1024, 8, 128
EPS = 2.220446049250313e-16
TBLK = 1024
NEG_INF = float("-inf")


def _moe_dense_body(x_ref, wg_ref, w_ref, b_ref, o_ref, wbf_ref):
    @pl.when(pl.program_id(0) == 0)
    def _():
        for e in range(E):
            wbf_ref[:, e * OUT:(e + 1) * OUT] = w_ref[e].astype(jnp.bfloat16)

    x = x_ref[...]
    xb = x.astype(jnp.bfloat16)
    logits = jnp.dot(xb, wg_ref[...].astype(jnp.bfloat16),
                     preferred_element_type=jnp.float32)
    m1 = jnp.max(logits, axis=1, keepdims=True)
    d = logits - m1
    is1 = d == 0.0
    masked = jnp.where(is1, NEG_INF, d)
    m2 = jnp.max(masked, axis=1, keepdims=True)
    t = jnp.exp(m2)
    gates = jnp.where(is1, 1.0, jnp.where(masked == m2, t, 0.0)) / (1.0 + t)
    acc = jnp.zeros((TBLK, OUT), jnp.float32)
    for e in range(E):
        o = jnp.dot(xb, wbf_ref[:, e * OUT:(e + 1) * OUT],
                    preferred_element_type=jnp.float32)
        o = o + b_ref[:, e * OUT:(e + 1) * OUT]
        acc = acc + gates[:, e:e + 1] * jnp.exp(o)
    acc = jnp.where(acc == 0.0, EPS, acc)
    o_ref[...] = jnp.log(acc)


def kernel(x, w_gate, W_exp, b_exp):
    return pl.pallas_call(
        _moe_dense_body,
        grid=(N // TBLK,),
        in_specs=[
            pl.BlockSpec((TBLK, D), lambda i: (i, 0)),
            pl.BlockSpec((D, E), lambda i: (0, 0)),
            pl.BlockSpec((E, D, OUT), lambda i: (0, 0, 0)),
            pl.BlockSpec((1, E * OUT), lambda i: (0, 0)),
        ],
        out_specs=pl.BlockSpec((TBLK, OUT), lambda i: (i, 0)),
        out_shape=jax.ShapeDtypeStruct((N, OUT), jnp.float32),
        scratch_shapes=[pltpu.VMEM((D, E * OUT), jnp.bfloat16)],
    )(x, w_gate, W_exp, b_exp.reshape(1, E * OUT))


# speedup vs baseline: 1.2778x; 1.2778x over previous
import jax
import jax.numpy as jnp
from jax import lax
from jax.experimental import pallas as pl
from jax.experimental.pallas import tpu as pltpu

N, D, E, OUT = 2048, 1024, 8, 128
EPS = 2.220446049250313e-16
TBLK = 1024
NEG_INF = float("-inf")


def _moe_dense_body(x_ref, wg_ref, w_ref, b_ref, o_ref, wbf_ref):
    @pl.when(pl.program_id(0) == 0)
    def _():
        for e in range(E):
            wbf_ref[:, e * OUT:(e + 1) * OUT] = w_ref[e].astype(jnp.bfloat16)

    x = x_ref[...]
    xb = x.astype(jnp.bfloat16)
    logits = jnp.dot(xb, wg_ref[...].astype(jnp.bfloat16),
                     preferred_element_type=jnp.float32)
    m1 = jnp.max(logits, axis=1, keepdims=True)
    d = logits - m1
    is1 = d == 0.0
    masked = jnp.where(is1, NEG_INF, d)
    m2 = jnp.max(masked, axis=1, keepdims=True)
    t = jnp.exp(m2)
    gates = jnp.where(is1, 1.0, jnp.where(masked == m2, t, 0.0)) / (1.0 + t)
    big = jnp.dot(xb, wbf_ref[...], preferred_element_type=jnp.float32)
    acc = jnp.zeros((TBLK, OUT), jnp.float32)
    for e in range(E):
        o = big[:, e * OUT:(e + 1) * OUT] + b_ref[:, e * OUT:(e + 1) * OUT]
        acc = acc + gates[:, e:e + 1] * jnp.exp(o)
    acc = jnp.where(acc == 0.0, EPS, acc)
    o_ref[...] = jnp.log(acc)


def kernel(x, w_gate, W_exp, b_exp):
    return pl.pallas_call(
        _moe_dense_body,
        grid=(N // TBLK,),
        in_specs=[
            pl.BlockSpec((TBLK, D), lambda i: (i, 0)),
            pl.BlockSpec((D, E), lambda i: (0, 0)),
            pl.BlockSpec((E, D, OUT), lambda i: (0, 0, 0)),
            pl.BlockSpec((1, E * OUT), lambda i: (0, 0)),
        ],
        out_specs=pl.BlockSpec((TBLK, OUT), lambda i: (i, 0)),
        out_shape=jax.ShapeDtypeStruct((N, OUT), jnp.float32),
        scratch_shapes=[pltpu.VMEM((D, E * OUT), jnp.bfloat16)],
    )(x, w_gate, W_exp, b_exp.reshape(1, E * OUT))


# speedup vs baseline: 1.4763x; 1.1554x over previous
import jax
import jax.numpy as jnp
from jax import lax
from jax.experimental import pallas as pl
from jax.experimental.pallas import tpu as pltpu

N, D, E, OUT = 2048, 1024, 8, 128
EPS = 2.220446049250313e-16
TBLK = 1024
NEG_INF = float("-inf")


def _moe_dense_body(x_ref, wgt_ref, w_ref, b_ref, o_ref, wbf_ref, wgbf_ref):
    @pl.when(pl.program_id(0) == 0)
    def _():
        for e in range(E):
            wbf_ref[:, e * OUT:(e + 1) * OUT] = w_ref[e].astype(jnp.bfloat16)
        wgbf_ref[...] = jnp.transpose(wgt_ref[...]).astype(jnp.bfloat16)

    x = x_ref[...]
    xb = x.astype(jnp.bfloat16)
    logits = jnp.dot(xb, wgbf_ref[...],
                     preferred_element_type=jnp.float32)
    m1 = jnp.max(logits, axis=1, keepdims=True)
    d = logits - m1
    is1 = d == 0.0
    masked = jnp.where(is1, NEG_INF, d)
    m2 = jnp.max(masked, axis=1, keepdims=True)
    t = jnp.exp(m2)
    gates = jnp.where(is1, 1.0, jnp.where(masked == m2, t, 0.0)) / (1.0 + t)
    big = jnp.dot(xb, wbf_ref[...], preferred_element_type=jnp.float32)
    acc = jnp.zeros((TBLK, OUT), jnp.float32)
    for e in range(E):
        o = big[:, e * OUT:(e + 1) * OUT] + b_ref[:, e * OUT:(e + 1) * OUT]
        acc = acc + gates[:, e:e + 1] * jnp.exp(o)
    acc = jnp.where(acc == 0.0, EPS, acc)
    o_ref[...] = jnp.log(acc)


def kernel(x, w_gate, W_exp, b_exp):
    return pl.pallas_call(
        _moe_dense_body,
        grid=(N // TBLK,),
        in_specs=[
            pl.BlockSpec((TBLK, D), lambda i: (i, 0)),
            pl.BlockSpec((E, D), lambda i: (0, 0)),
            pl.BlockSpec((E, D, OUT), lambda i: (0, 0, 0)),
            pl.BlockSpec((1, E * OUT), lambda i: (0, 0)),
        ],
        out_specs=pl.BlockSpec((TBLK, OUT), lambda i: (i, 0)),
        out_shape=jax.ShapeDtypeStruct((N, OUT), jnp.float32),
        scratch_shapes=[
            pltpu.VMEM((D, E * OUT), jnp.bfloat16),
            pltpu.VMEM((D, E), jnp.bfloat16),
        ],
    )(x, w_gate.T, W_exp, b_exp.reshape(1, E * OUT))


# speedup vs baseline: 1.5153x; 1.0264x over previous
import jax
import jax.numpy as jnp
from jax import lax
from jax.experimental import pallas as pl
from jax.experimental.pallas import tpu as pltpu

N, D, E, OUT = 2048, 1024, 8, 128
EPS = 2.220446049250313e-16
TBLK = 512
NEG_INF = float("-inf")


def _moe_dense_body(x_ref, wgt_ref, w_ref, b_ref, o_ref, wbf_ref, wgbf_ref):
    @pl.when(pl.program_id(0) == 0)
    def _():
        for e in range(E):
            wbf_ref[:, e * OUT:(e + 1) * OUT] = w_ref[e].astype(jnp.bfloat16)
        wgbf_ref[...] = jnp.transpose(wgt_ref[...]).astype(jnp.bfloat16)

    x = x_ref[...]
    xb = x.astype(jnp.bfloat16)
    logits = jnp.dot(xb, wgbf_ref[...],
                     preferred_element_type=jnp.float32)
    m1 = jnp.max(logits, axis=1, keepdims=True)
    d = logits - m1
    is1 = d == 0.0
    masked = jnp.where(is1, NEG_INF, d)
    m2 = jnp.max(masked, axis=1, keepdims=True)
    t = jnp.exp(m2)
    gates = jnp.where(is1, 1.0, jnp.where(masked == m2, t, 0.0)) / (1.0 + t)
    big = jnp.dot(xb, wbf_ref[...], preferred_element_type=jnp.float32)
    acc = jnp.zeros((TBLK, OUT), jnp.float32)
    for e in range(E):
        o = big[:, e * OUT:(e + 1) * OUT] + b_ref[:, e * OUT:(e + 1) * OUT]
        acc = acc + gates[:, e:e + 1] * jnp.exp(o)
    acc = jnp.where(acc == 0.0, EPS, acc)
    o_ref[...] = jnp.log(acc)


def kernel(x, w_gate, W_exp, b_exp):
    return pl.pallas_call(
        _moe_dense_body,
        grid=(N // TBLK,),
        in_specs=[
            pl.BlockSpec((TBLK, D), lambda i: (i, 0)),
            pl.BlockSpec((E, D), lambda i: (0, 0)),
            pl.BlockSpec((E, D, OUT), lambda i: (0, 0, 0)),
            pl.BlockSpec((1, E * OUT), lambda i: (0, 0)),
        ],
        out_specs=pl.BlockSpec((TBLK, OUT), lambda i: (i, 0)),
        out_shape=jax.ShapeDtypeStruct((N, OUT), jnp.float32),
        scratch_shapes=[
            pltpu.VMEM((D, E * OUT), jnp.bfloat16),
            pltpu.VMEM((D, E), jnp.bfloat16),
        ],
    )(x, w_gate.T, W_exp, b_exp.reshape(1, E * OUT))
